# baseline (speedup 1.0000x reference)
import jax
import jax.numpy as jnp
import numpy as np
from jax.experimental import pallas as pl

B, T, C = 2, 2048, 768
E, MIN_E, HD = 16, 2, 64
BLK = 256


def _fold_sum(x):
    while x.shape[1] > HD:
        h = x.shape[1] // 2
        x = x[:, :h] + x[:, h:]
    return x


def _gate_qkv_body(x_ref, sim_ref, gates_ref, w_all_ref,
                   q_ref, k_ref, v_ref, wgt_ref):
    x = x_ref[...]
    xnorm = jnp.sqrt(jnp.sum(x * x, axis=1, keepdims=True))
    hn = x / jnp.maximum(xnorm, 1e-12)
    sim = sim_ref[...]
    snorm = jnp.sqrt(jnp.sum(sim * sim, axis=0, keepdims=True))
    sn = sim / jnp.maximum(snorm, 1e-12)
    logits = jnp.dot(hn, sn, preferred_element_type=jnp.float32)
    logits = logits - jax.nn.sigmoid(gates_ref[...])
    gated = jnp.maximum(logits, 0.0)
    mask = (gated > 0.0).astype(jnp.float32)
    inactive = jnp.sum(mask, axis=1, keepdims=True) == 0.0
    iota = jax.lax.broadcasted_iota(jnp.int32, logits.shape, 1)
    max1 = jnp.max(logits, axis=1, keepdims=True)
    idx1 = jnp.min(jnp.where(logits == max1, iota, E), axis=1, keepdims=True)
    l2 = jnp.where(iota == idx1, -jnp.inf, logits)
    max2 = jnp.max(l2, axis=1, keepdims=True)
    idx2 = jnp.min(jnp.where(l2 == max2, iota, E), axis=1, keepdims=True)
    fb = jnp.logical_or(iota == idx1, iota == idx2)
    mask = jnp.where(jnp.logical_and(inactive, fb), 1.0, mask)
    gm = jnp.where(mask > 0.0, gated, jnp.float32(-1e9))
    gm_max = jnp.max(gm, axis=1, keepdims=True)
    p = jnp.exp(gm - gm_max)
    probs = p / jnp.sum(p, axis=1, keepdims=True)
    wgt_ref[...] = probs * mask
    P = jnp.dot(x.astype(jnp.bfloat16), w_all_ref[...],
                preferred_element_type=jnp.float32)
    mexp = jnp.concatenate(
        [jnp.broadcast_to(mask[:, e:e + 1], (BLK, HD)) for e in range(E)],
        axis=1)
    q_ref[...] = _fold_sum(P[:, :E * HD] * mexp)
    k_ref[...] = _fold_sum(P[:, E * HD:2 * E * HD] * mexp)
    v_ref[...] = _fold_sum(P[:, 2 * E * HD:] * mexp)


def _attn_out_body(q_ref, k_ref, v_ref, wgt_ref, ost_ref, o_ref):
    qi = pl.program_id(1)
    q = q_ref[0].astype(jnp.bfloat16)
    scale = jnp.float32(1.0 / np.sqrt(HD))

    def _block(j, carry, masked):
        acc, m_run, l_run = carry
        k = k_ref[0, pl.ds(j * BLK, BLK), :].astype(jnp.bfloat16)
        v = v_ref[0, pl.ds(j * BLK, BLK), :].astype(jnp.bfloat16)
        s = jax.lax.dot_general(q, k, (((1,), (1,)), ((), ())),
                                preferred_element_type=jnp.float32) * scale
        if masked:
            r = jax.lax.broadcasted_iota(jnp.int32, s.shape, 0)
            c = jax.lax.broadcasted_iota(jnp.int32, s.shape, 1)
            s = jnp.where(c <= r, s, jnp.float32(-1e9))
        m_new = jnp.maximum(m_run, jnp.max(s, axis=1, keepdims=True))
        alpha = jnp.exp(m_run - m_new)
        p = jnp.exp(s - m_new)
        l_new = l_run * alpha + jnp.sum(p, axis=1, keepdims=True)
        acc = acc * alpha + jnp.dot(p.astype(jnp.bfloat16), v,
                                    preferred_element_type=jnp.float32)
        return acc, m_new, l_new

    init = (jnp.zeros((BLK, HD), jnp.float32),
            jnp.full((BLK, 1), -1e30, jnp.float32),
            jnp.zeros((BLK, 1), jnp.float32))
    carry = jax.lax.fori_loop(0, qi, lambda j, c: _block(j, c, False), init)
    acc, _, l_run = _block(qi, carry, True)
    o = acc / l_run
    w = wgt_ref[0]
    a = jnp.concatenate([w[:, e:e + 1] * o for e in range(E)], axis=1)
    o_ref[0] = jnp.dot(a.astype(jnp.bfloat16), ost_ref[...],
                       preferred_element_type=jnp.float32)


def kernel(hidden_states, sim_matrix, gates, q_proj, k_proj, v_proj, o_proj):
    flat = hidden_states.reshape(B * T, C)
    w_all = jnp.concatenate(
        [q_proj.transpose(1, 0, 2).reshape(C, E * HD),
         k_proj.transpose(1, 0, 2).reshape(C, E * HD),
         v_proj.transpose(1, 0, 2).reshape(C, E * HD)],
        axis=1).astype(jnp.bfloat16)
    gates2 = gates.reshape(1, E)
    nblk = (B * T) // BLK

    q, k, v, wgt = pl.pallas_call(
        _gate_qkv_body,
        grid=(nblk,),
        in_specs=[
            pl.BlockSpec((BLK, C), lambda i: (i, 0)),
            pl.BlockSpec((C, E), lambda i: (0, 0)),
            pl.BlockSpec((1, E), lambda i: (0, 0)),
            pl.BlockSpec((C, 3 * E * HD), lambda i: (0, 0)),
        ],
        out_specs=[
            pl.BlockSpec((BLK, HD), lambda i: (i, 0)),
            pl.BlockSpec((BLK, HD), lambda i: (i, 0)),
            pl.BlockSpec((BLK, HD), lambda i: (i, 0)),
            pl.BlockSpec((BLK, E), lambda i: (i, 0)),
        ],
        out_shape=[
            jax.ShapeDtypeStruct((B * T, HD), jnp.float32),
            jax.ShapeDtypeStruct((B * T, HD), jnp.float32),
            jax.ShapeDtypeStruct((B * T, HD), jnp.float32),
            jax.ShapeDtypeStruct((B * T, E), jnp.float32),
        ],
    )(flat, sim_matrix, gates2, w_all)

    q3 = q.reshape(B, T, HD)
    k3 = k.reshape(B, T, HD)
    v3 = v.reshape(B, T, HD)
    w3 = wgt.reshape(B, T, E)
    o_st = o_proj.reshape(E * HD, C).astype(jnp.bfloat16)

    out = pl.pallas_call(
        _attn_out_body,
        grid=(B, T // BLK),
        in_specs=[
            pl.BlockSpec((1, BLK, HD), lambda b, i: (b, i, 0)),
            pl.BlockSpec((1, T, HD), lambda b, i: (b, 0, 0)),
            pl.BlockSpec((1, T, HD), lambda b, i: (b, 0, 0)),
            pl.BlockSpec((1, BLK, E), lambda b, i: (b, i, 0)),
            pl.BlockSpec((E * HD, C), lambda b, i: (0, 0)),
        ],
        out_specs=pl.BlockSpec((1, BLK, C), lambda b, i: (b, i, 0)),
        out_shape=jax.ShapeDtypeStruct((B, T, C), jnp.float32),
    )(q3, k3, v3, w3, o_st)

    return out


# speedup vs baseline: 1.3443x; 1.3443x over previous
import jax
import jax.numpy as jnp
import numpy as np
from jax.experimental import pallas as pl
from jax.experimental.pallas import tpu as pltpu

B, T, C = 2, 2048, 768
E, MIN_E, HD = 16, 2, 64
BLK = 256


def _fold_sum(x):
    while x.shape[1] > HD:
        h = x.shape[1] // 2
        x = x[:, :h] + x[:, h:]
    return x


def _body(x_ref, sim_ref, gates_ref, w_all_ref, ost_ref, out_ref,
          kscr, vscr):
    qi = pl.program_id(1)

    @pl.when(jnp.logical_and(pl.program_id(0) == 0, qi == 0))
    def _init():
        vscr[...] = jnp.zeros_like(vscr)

    x = x_ref[...]
    xnorm = jnp.sqrt(jnp.sum(x * x, axis=1, keepdims=True))
    hn = x / jnp.maximum(xnorm, 1e-12)
    sim = sim_ref[...]
    snorm = jnp.sqrt(jnp.sum(sim * sim, axis=0, keepdims=True))
    sn = sim / jnp.maximum(snorm, 1e-12)
    logits = jnp.dot(hn, sn, preferred_element_type=jnp.float32)
    logits = logits - jax.nn.sigmoid(gates_ref[...])
    gated = jnp.maximum(logits, 0.0)
    mask = (gated > 0.0).astype(jnp.float32)
    inactive = jnp.sum(mask, axis=1, keepdims=True) == 0.0
    iota = jax.lax.broadcasted_iota(jnp.int32, logits.shape, 1)
    max1 = jnp.max(logits, axis=1, keepdims=True)
    idx1 = jnp.min(jnp.where(logits == max1, iota, E), axis=1, keepdims=True)
    l2 = jnp.where(iota == idx1, -jnp.inf, logits)
    max2 = jnp.max(l2, axis=1, keepdims=True)
    idx2 = jnp.min(jnp.where(l2 == max2, iota, E), axis=1, keepdims=True)
    fb = jnp.logical_or(iota == idx1, iota == idx2)
    mask = jnp.where(jnp.logical_and(inactive, fb), 1.0, mask)
    gm = jnp.where(mask > 0.0, gated, jnp.float32(-1e9))
    gm_max = jnp.max(gm, axis=1, keepdims=True)
    pw = jnp.exp(gm - gm_max)
    w = (pw / jnp.sum(pw, axis=1, keepdims=True)) * mask
    P = jnp.dot(x.astype(jnp.bfloat16), w_all_ref[...],
                preferred_element_type=jnp.float32)
    mexp = jnp.concatenate(
        [jnp.broadcast_to(mask[:, e:e + 1], (BLK, HD)) for e in range(E)],
        axis=1)
    q = _fold_sum(P[:, :E * HD] * mexp)
    k = _fold_sum(P[:, E * HD:2 * E * HD] * mexp)
    v = _fold_sum(P[:, 2 * E * HD:] * mexp)
    kscr[pl.ds(qi * BLK, BLK), :] = k.astype(jnp.bfloat16)
    vscr[pl.ds(qi * BLK, BLK), :] = v.astype(jnp.bfloat16)
    scale = jnp.float32(1.0 / np.sqrt(HD))
    s = jax.lax.dot_general(q.astype(jnp.bfloat16), kscr[...],
                            (((1,), (1,)), ((), ())),
                            preferred_element_type=jnp.float32) * scale
    r = qi * BLK + jax.lax.broadcasted_iota(jnp.int32, s.shape, 0)
    c = jax.lax.broadcasted_iota(jnp.int32, s.shape, 1)
    s = jnp.where(c <= r, s, jnp.float32(-1e9))
    m = jnp.max(s, axis=1, keepdims=True)
    p = jnp.exp(s - m)
    o = jnp.dot(p.astype(jnp.bfloat16), vscr[...],
                preferred_element_type=jnp.float32)
    o = o / jnp.sum(p, axis=1, keepdims=True)
    a = jnp.concatenate([w[:, e:e + 1] * o for e in range(E)], axis=1)
    out_ref[...] = jnp.dot(a.astype(jnp.bfloat16), ost_ref[...],
                           preferred_element_type=jnp.float32)


def kernel(hidden_states, sim_matrix, gates, q_proj, k_proj, v_proj, o_proj):
    flat = hidden_states.reshape(B * T, C)
    w_all = jnp.concatenate(
        [q_proj.transpose(1, 0, 2).reshape(C, E * HD),
         k_proj.transpose(1, 0, 2).reshape(C, E * HD),
         v_proj.transpose(1, 0, 2).reshape(C, E * HD)],
        axis=1).astype(jnp.bfloat16)
    gates2 = gates.reshape(1, E)
    o_st = o_proj.reshape(E * HD, C).astype(jnp.bfloat16)
    nq = T // BLK

    out = pl.pallas_call(
        _body,
        grid=(B, nq),
        in_specs=[
            pl.BlockSpec((BLK, C), lambda b, i: (b * nq + i, 0)),
            pl.BlockSpec((C, E), lambda b, i: (0, 0)),
            pl.BlockSpec((1, E), lambda b, i: (0, 0)),
            pl.BlockSpec((C, 3 * E * HD), lambda b, i: (0, 0)),
            pl.BlockSpec((E * HD, C), lambda b, i: (0, 0)),
        ],
        out_specs=pl.BlockSpec((BLK, C), lambda b, i: (b * nq + i, 0)),
        out_shape=jax.ShapeDtypeStruct((B * T, C), jnp.float32),
        scratch_shapes=[
            pltpu.VMEM((T, HD), jnp.bfloat16),
            pltpu.VMEM((T, HD), jnp.bfloat16),
        ],
    )(flat, sim_matrix, gates2, w_all, o_st)

    return out.reshape(B, T, C)
